# baseline (speedup 1.0000x reference)
import jax
import jax.numpy as jnp
from jax.experimental import pallas as pl

N = 10000
G = 64
EPS_BN = 1e-5


def _gat_conv(x, W, att_src, att_dst, bias, src, dst):
    h = x @ W
    a_s = h @ att_src
    a_d = h @ att_dst
    alpha = a_s[src] + a_d[dst]
    alpha = jnp.where(alpha > 0, alpha, 0.2 * alpha)
    amax = jax.ops.segment_max(alpha, dst, num_segments=N)
    amax = jnp.where(jnp.isfinite(amax), amax, 0.0)
    ex = jnp.exp(alpha - amax[dst])
    denom = jax.ops.segment_sum(ex, dst, num_segments=N)
    coef = ex / (denom[dst] + 1e-16)
    out = jax.ops.segment_sum(h[src] * coef[:, None], dst, num_segments=N)
    return out + bias


def _bn(x, gamma, beta):
    mu = jnp.mean(x, axis=0)
    var = jnp.mean((x - mu) ** 2, axis=0)
    return (x - mu) / jnp.sqrt(var + EPS_BN) * gamma + beta


def _mlp_kernel(p_ref, w1_ref, b1_ref, w2_ref, b2_ref, w3_ref, b3_ref, w4_ref, b4_ref, o_ref):
    z = jax.nn.relu(p_ref[...] @ w1_ref[...] + b1_ref[...])
    z = jax.nn.relu(z @ w2_ref[...] + b2_ref[...])
    z = jax.nn.relu(z @ w3_ref[...] + b3_ref[...])
    o_ref[...] = z @ w4_ref[...] + b4_ref[...]


def kernel(x, edge_index, batch, W1, a_src1, a_dst1, b1, W2, a_src2, a_dst2, b2, gamma, beta, m1w, m1b, m2w, m2b, m3w, m3b, m4w, m4b):
    src, dst = edge_index[0], edge_index[1]
    h = _gat_conv(x, W1, a_src1, a_dst1, b1, src, dst)
    h = jax.nn.relu(h)
    h = _bn(h, gamma, beta)
    h = _gat_conv(h, W2, a_src2, a_dst2, b2, src, dst)
    h = jax.nn.relu(h)
    h = _bn(h, gamma, beta)
    sums = jax.ops.segment_sum(h, batch, num_segments=G)
    cnt = jax.ops.segment_sum(jnp.ones((N,), jnp.float32), batch, num_segments=G)
    pooled = sums / jnp.maximum(cnt, 1.0)[:, None]
    z = pl.pallas_call(
        _mlp_kernel,
        out_shape=jax.ShapeDtypeStruct((G, m4w.shape[1]), jnp.float32),
    )(pooled, m1w, m1b.reshape(1, -1), m2w, m2b.reshape(1, -1),
      m3w, m3b.reshape(1, -1), m4w, m4b.reshape(1, -1))
    return z


# speedup vs baseline: 28.8884x; 28.8884x over previous
import functools

import jax
import jax.numpy as jnp
from jax import lax
from jax.experimental import pallas as pl
from jax.experimental.pallas import tpu as pltpu
from jax.experimental.pallas import tpu_sc as plsc

N = 10000
E = 320000
D = 128
G = 64
EPS_BN = 1e-5

NC = 2
NS = 16
L = 16
NW = NC * NS

HR = E // 64
HPT = 160
PADH = NW * HPT
CR = 16
CH = HPT // CR
NPT = N // NS


def _sc_edge(h, a_s, a_d, src2, dst2):
    mesh = plsc.VectorSubcoreMesh(core_axis_name="c", subcore_axis_name="s")

    @functools.partial(
        pl.kernel,
        out_type=(
            jax.ShapeDtypeStruct((NC, N, D), jnp.float32),
            jax.ShapeDtypeStruct((NW, N), jnp.float32),
        ),
        mesh=mesh,
        compiler_params=pltpu.CompilerParams(
            use_tc_tiling_on_sc=False, needs_layout_passes=False),
        scratch_types=[
            pltpu.VMEM((N,), jnp.float32),
            pltpu.VMEM((N,), jnp.float32),
            pltpu.VMEM((N,), jnp.float32),
            pltpu.VMEM((CR, 64), jnp.int32),
            pltpu.VMEM((CR, 64), jnp.int32),
            pltpu.VMEM((64, D), jnp.float32),
            pltpu.VMEM_SHARED((N, D), jnp.float32),
            pltpu.SemaphoreType.DMA,
        ],
    )
    def k(h_hbm, as_hbm, ad_hbm, src_hbm, dst_hbm, outp_hbm, denomp_hbm,
          as_v, ad_v, den_v, src_v, dst_v, rows_v, acc_sh, sem):
        cid = lax.axis_index("c")
        sid = lax.axis_index("s")
        wid = sid * NC + cid
        base = wid * HPT

        pltpu.sync_copy(as_hbm, as_v)
        pltpu.sync_copy(ad_hbm, ad_v)

        big = jnp.full((L,), -3e38, jnp.float32)

        def mx_body(i, carry):
            ma, md = carry
            ma = jnp.maximum(ma, as_v[pl.ds(i * L, L)])
            md = jnp.maximum(md, ad_v[pl.ds(i * L, L)])
            return ma, md

        ma, md = lax.fori_loop(0, N // L, mx_body, (big, big))
        msum = jnp.max(ma) + jnp.max(md)
        m = jnp.where(msum > 0, msum, 0.2 * msum)

        zz = jnp.zeros((L,), jnp.float32)

        def zden_body(i, _):
            den_v[pl.ds(i * L, L)] = zz
            return 0

        lax.fori_loop(0, N // L, zden_body, 0)

        def zrow_body(i, _):
            for f in range(D // L):
                rows_v[i, pl.ds(f * L, L)] = zz
            return 0

        lax.fori_loop(0, 64, zrow_body, 0)

        rbase = sid * NPT
        for c in range(9):
            pltpu.sync_copy(rows_v, acc_sh.at[pl.ds(rbase + c * 64, 64)])
        pltpu.sync_copy(rows_v.at[pl.ds(0, NPT - 576)],
                        acc_sh.at[pl.ds(rbase + 576, NPT - 576)])
        plsc.subcore_barrier()

        def chunk_body(c, _):
            cbase = base + c * CR
            pltpu.sync_copy(src_hbm.at[pl.ds(cbase, CR)], src_v)
            pltpu.sync_copy(dst_hbm.at[pl.ds(cbase, CR)], dst_v)

            def step_body(r, _):
                @pl.when(cbase + r < HR)
                def _():
                    pltpu.async_copy(h_hbm.at[src_v.at[r]], rows_v, sem).wait()

                    def group_body(g, _):
                        si = src_v[r, pl.ds(g * L, L)]
                        di = dst_v[r, pl.ds(g * L, L)]
                        al = (plsc.load_gather(as_v, [si])
                              + plsc.load_gather(ad_v, [di]))
                        al = jnp.where(al > 0, al, 0.2 * al)
                        exv = jnp.exp(al - m)
                        plsc.addupdate_scatter(den_v, [di], exv)
                        for j in range(L):
                            s = exv[j]
                            i = g * L + j
                            for f in range(D // L):
                                rows_v[i, pl.ds(f * L, L)] = (
                                    rows_v[i, pl.ds(f * L, L)] * s)
                        return 0

                    lax.fori_loop(0, 64 // L, group_body, 0)

                    pltpu.sync_copy(rows_v, acc_sh.at[dst_v.at[r]], add=True)
                return 0

            lax.fori_loop(0, CR, step_body, 0)
            return 0

        lax.fori_loop(0, CH, chunk_body, 0)

        pltpu.sync_copy(den_v, denomp_hbm.at[wid])
        plsc.subcore_barrier()

        for c in range(9):
            pltpu.sync_copy(acc_sh.at[pl.ds(rbase + c * 64, 64)],
                            outp_hbm.at[cid, pl.ds(rbase + c * 64, 64)])
        pltpu.sync_copy(acc_sh.at[pl.ds(rbase + 576, NPT - 576)],
                        outp_hbm.at[cid, pl.ds(rbase + 576, NPT - 576)])

    return k(h, a_s, a_d, src2, dst2)


def _dot(a, b):
    return jnp.dot(a, b)


def _rinv(x):
    r = 1.0 / x
    r = r * (2.0 - x * r)
    r = r * (2.0 - x * r)
    return r


def _rsqrtp(x):
    s = jnp.sqrt(x)
    s = 0.5 * (s + x * _rinv(s))
    return _rinv(s)


def _tc_in_body(x_ref, w_ref, a_ref, h_ref, aa_ref):
    h = _dot(x_ref[...], w_ref[...])
    h_ref[...] = h
    aa_ref[...] = _dot(h, a_ref[...])


def _tc_mid_body(outp_ref, denomt_ref, b_ref, gamma_ref, beta_ref, w_ref, a_ref,
                 h_ref, aa_ref):
    out = outp_ref[0] + outp_ref[1]
    denom = jnp.sum(denomt_ref[...], axis=1, keepdims=True)
    h = out * _rinv(denom + 1e-16) + b_ref[...]
    h = jnp.maximum(h, 0.0)
    mu = jnp.mean(h, axis=0, keepdims=True)
    var = jnp.mean((h - mu) ** 2, axis=0, keepdims=True)
    xn = (h - mu) * _rsqrtp(var + EPS_BN) * gamma_ref[...] + beta_ref[...]
    h2 = _dot(xn, w_ref[...])
    h_ref[...] = h2
    aa_ref[...] = _dot(h2, a_ref[...])


def _tc_fin_body(outp_ref, denomt_ref, b_ref, gamma_ref, beta_ref, batch_ref,
                 w1_ref, b1_ref, w2_ref, b2_ref, w3_ref, b3_ref, w4_ref, b4_ref,
                 o_ref):
    out = outp_ref[0] + outp_ref[1]
    denom = jnp.sum(denomt_ref[...], axis=1, keepdims=True)
    h = out * _rinv(denom + 1e-16) + b_ref[...]
    h = jnp.maximum(h, 0.0)
    mu = jnp.mean(h, axis=0, keepdims=True)
    var = jnp.mean((h - mu) ** 2, axis=0, keepdims=True)
    h = (h - mu) * _rsqrtp(var + EPS_BN) * gamma_ref[...] + beta_ref[...]
    rows = lax.broadcasted_iota(jnp.int32, (G, N), 0)
    oh = (rows == batch_ref[...]).astype(jnp.float32)
    sums = jnp.dot(oh, h, precision=lax.Precision.HIGHEST)
    cnt = jnp.sum(oh, axis=1, keepdims=True)
    pooled = sums * _rinv(jnp.maximum(cnt, 1.0))
    z = jnp.maximum(_dot(pooled, w1_ref[...]) + b1_ref[...], 0.0)
    z = jnp.maximum(_dot(z, w2_ref[...]) + b2_ref[...], 0.0)
    z = jnp.maximum(_dot(z, w3_ref[...]) + b3_ref[...], 0.0)
    o_ref[...] = _dot(z, w4_ref[...]) + b4_ref[...]


def kernel(x, edge_index, batch, W1, a_src1, a_dst1, b1, W2, a_src2, a_dst2, b2,
           gamma, beta, m1w, m1b, m2w, m2b, m3w, m3b, m4w, m4b):
    src, dst = edge_index[0], edge_index[1]
    src2 = jnp.pad(src.reshape(HR, 64), ((0, PADH - HR), (0, 0)))
    dst2 = jnp.pad(dst.reshape(HR, 64), ((0, PADH - HR), (0, 0)))

    acat1 = jnp.stack([a_src1, a_dst1], axis=1)
    acat2 = jnp.stack([a_src2, a_dst2], axis=1)

    h1, aa1 = pl.pallas_call(
        _tc_in_body,
        out_shape=(jax.ShapeDtypeStruct((N, D), jnp.float32),
                   jax.ShapeDtypeStruct((N, 2), jnp.float32)),
    )(x, W1, acat1)

    outp1, denomp1 = _sc_edge(h1, aa1[:, 0], aa1[:, 1], src2, dst2)

    h2, aa2 = pl.pallas_call(
        _tc_mid_body,
        out_shape=(jax.ShapeDtypeStruct((N, D), jnp.float32),
                   jax.ShapeDtypeStruct((N, 2), jnp.float32)),
    )(outp1, denomp1.T, b1.reshape(1, D), gamma.reshape(1, D),
      beta.reshape(1, D), W2, acat2)

    outp2, denomp2 = _sc_edge(h2, aa2[:, 0], aa2[:, 1], src2, dst2)

    z = pl.pallas_call(
        _tc_fin_body,
        out_shape=jax.ShapeDtypeStruct((G, m4w.shape[1]), jnp.float32),
    )(outp2, denomp2.T, b2.reshape(1, D), gamma.reshape(1, D),
      beta.reshape(1, D), batch.reshape(1, N),
      m1w, m1b.reshape(1, -1), m2w, m2b.reshape(1, -1),
      m3w, m3b.reshape(1, -1), m4w, m4b.reshape(1, -1))
    return z


# speedup vs baseline: 41.3264x; 1.4306x over previous
import functools

import jax
import jax.numpy as jnp
from jax import lax
from jax.experimental import pallas as pl
from jax.experimental.pallas import tpu as pltpu
from jax.experimental.pallas import tpu_sc as plsc

N = 10000
E = 320000
D = 128
G = 64
EPS_BN = 1e-5

NC = 2
NS = 16
L = 16
NW = NC * NS

HR = E // 64
HPT = 160
PADH = NW * HPT
CR = 16
CH = HPT // CR
NPT = N // NS


def _sc_edge(h, a_s, a_d, src2, dst2):
    mesh = plsc.VectorSubcoreMesh(core_axis_name="c", subcore_axis_name="s")

    @functools.partial(
        pl.kernel,
        out_type=(
            jax.ShapeDtypeStruct((NC, N, D), jnp.float32),
            jax.ShapeDtypeStruct((NW, N), jnp.float32),
        ),
        mesh=mesh,
        compiler_params=pltpu.CompilerParams(
            use_tc_tiling_on_sc=False, needs_layout_passes=False),
        scratch_types=[
            pltpu.VMEM((N,), jnp.float32),
            pltpu.VMEM((N,), jnp.float32),
            pltpu.VMEM((N,), jnp.float32),
            pltpu.VMEM((CR, 64), jnp.int32),
            pltpu.VMEM((CR, 64), jnp.int32),
            pltpu.VMEM((2, 64, D), jnp.float32),
            pltpu.VMEM_SHARED((N, D), jnp.float32),
            pltpu.SemaphoreType.DMA,
            pltpu.SemaphoreType.DMA,
        ],
    )
    def k(h_hbm, as_hbm, ad_hbm, src_hbm, dst_hbm, outp_hbm, denomp_hbm,
          as_v, ad_v, den_v, src_v, dst_v, rows2_v, acc_sh, gsem0, gsem1):
        rows_v = rows2_v.at[0]
        cid = lax.axis_index("c")
        sid = lax.axis_index("s")
        wid = sid * NC + cid
        base = wid * HPT

        pltpu.sync_copy(as_hbm, as_v)
        pltpu.sync_copy(ad_hbm, ad_v)

        big = jnp.full((L,), -3e38, jnp.float32)

        def mx_body(i, carry):
            ma, md = carry
            ma = jnp.maximum(ma, as_v[pl.ds(i * L, L)])
            md = jnp.maximum(md, ad_v[pl.ds(i * L, L)])
            return ma, md

        ma, md = lax.fori_loop(0, N // L, mx_body, (big, big))
        msum = jnp.max(ma) + jnp.max(md)
        m = jnp.where(msum > 0, msum, 0.2 * msum)

        zz = jnp.zeros((L,), jnp.float32)

        def zden_body(i, _):
            den_v[pl.ds(i * L, L)] = zz
            return 0

        lax.fori_loop(0, N // L, zden_body, 0)

        def zrow_body(i, _):
            for f in range(D // L):
                rows_v[i, pl.ds(f * L, L)] = zz
            return 0

        lax.fori_loop(0, 64, zrow_body, 0)

        rbase = sid * NPT
        for c in range(9):
            pltpu.sync_copy(rows_v, acc_sh.at[pl.ds(rbase + c * 64, 64)])
        pltpu.sync_copy(rows_v.at[pl.ds(0, NPT - 576)],
                        acc_sh.at[pl.ds(rbase + 576, NPT - 576)])
        plsc.subcore_barrier()

        gsems = (gsem0, gsem1)

        def fire(r, b):
            pltpu.async_copy(h_hbm.at[src_v.at[r]], rows2_v.at[b], gsems[b])

        def wait(r, b):
            pltpu.make_async_copy(h_hbm.at[src_v.at[r]], rows2_v.at[b],
                                  gsems[b]).wait()

        def compute_and_scatter(r, b):
            buf = rows2_v.at[b]

            def group_body(g, _):
                si = src_v[r, pl.ds(g * L, L)]
                di = dst_v[r, pl.ds(g * L, L)]
                al = (plsc.load_gather(as_v, [si])
                      + plsc.load_gather(ad_v, [di]))
                al = jnp.where(al > 0, al, 0.2 * al)
                exv = jnp.exp(al - m)
                plsc.addupdate_scatter(den_v, [di], exv)
                for j in range(L):
                    s = exv[j]
                    i = g * L + j
                    for f in range(D // L):
                        buf[i, pl.ds(f * L, L)] = buf[i, pl.ds(f * L, L)] * s
                return 0

            lax.fori_loop(0, 64 // L, group_body, 0)

            pltpu.sync_copy(buf, acc_sh.at[dst_v.at[r]], add=True)

        def chunk_body(c, _):
            cbase = base + c * CR
            pltpu.sync_copy(src_hbm.at[pl.ds(cbase, CR)], src_v)
            pltpu.sync_copy(dst_hbm.at[pl.ds(cbase, CR)], dst_v)

            @pl.when(cbase < HR)
            def _():
                fire(0, 0)

            def pair_body(p, _):
                r0 = 2 * p
                for b in range(2):
                    r = r0 + b

                    @pl.when(cbase + r < HR)
                    def _():
                        wait(r, b)
                        nxt = r + 1
                        @pl.when((nxt < CR) & (cbase + nxt < HR))
                        def _():
                            fire(nxt, 1 - b)
                        compute_and_scatter(r, b)
                return 0

            lax.fori_loop(0, CR // 2, pair_body, 0)
            return 0

        lax.fori_loop(0, CH, chunk_body, 0)

        pltpu.sync_copy(den_v, denomp_hbm.at[wid])
        plsc.subcore_barrier()

        for c in range(9):
            pltpu.sync_copy(acc_sh.at[pl.ds(rbase + c * 64, 64)],
                            outp_hbm.at[cid, pl.ds(rbase + c * 64, 64)])
        pltpu.sync_copy(acc_sh.at[pl.ds(rbase + 576, NPT - 576)],
                        outp_hbm.at[cid, pl.ds(rbase + 576, NPT - 576)])

    return k(h, a_s, a_d, src2, dst2)


def _dot(a, b):
    return jnp.dot(a, b)


def _rinv(x):
    r = 1.0 / x
    r = r * (2.0 - x * r)
    r = r * (2.0 - x * r)
    return r


def _rsqrtp(x):
    s = jnp.sqrt(x)
    s = 0.5 * (s + x * _rinv(s))
    return _rinv(s)


def _tc_in_body(x_ref, w_ref, a_ref, h_ref, aa_ref):
    h = _dot(x_ref[...], w_ref[...])
    h_ref[...] = h
    aa_ref[...] = _dot(h, a_ref[...])


def _tc_mid_body(outp_ref, denomt_ref, b_ref, gamma_ref, beta_ref, w_ref, a_ref,
                 h_ref, aa_ref):
    out = outp_ref[0] + outp_ref[1]
    denom = jnp.sum(denomt_ref[...], axis=1, keepdims=True)
    h = out * _rinv(denom + 1e-16) + b_ref[...]
    h = jnp.maximum(h, 0.0)
    mu = jnp.mean(h, axis=0, keepdims=True)
    var = jnp.mean((h - mu) ** 2, axis=0, keepdims=True)
    xn = (h - mu) * _rsqrtp(var + EPS_BN) * gamma_ref[...] + beta_ref[...]
    h2 = _dot(xn, w_ref[...])
    h_ref[...] = h2
    aa_ref[...] = _dot(h2, a_ref[...])


def _tc_fin_body(outp_ref, denomt_ref, b_ref, gamma_ref, beta_ref, batch_ref,
                 w1_ref, b1_ref, w2_ref, b2_ref, w3_ref, b3_ref, w4_ref, b4_ref,
                 o_ref):
    out = outp_ref[0] + outp_ref[1]
    denom = jnp.sum(denomt_ref[...], axis=1, keepdims=True)
    h = out * _rinv(denom + 1e-16) + b_ref[...]
    h = jnp.maximum(h, 0.0)
    mu = jnp.mean(h, axis=0, keepdims=True)
    var = jnp.mean((h - mu) ** 2, axis=0, keepdims=True)
    h = (h - mu) * _rsqrtp(var + EPS_BN) * gamma_ref[...] + beta_ref[...]
    rows = lax.broadcasted_iota(jnp.int32, (G, N), 0)
    oh = (rows == batch_ref[...]).astype(jnp.float32)
    sums = jnp.dot(oh, h, precision=lax.Precision.HIGHEST)
    cnt = jnp.sum(oh, axis=1, keepdims=True)
    pooled = sums * _rinv(jnp.maximum(cnt, 1.0))
    z = jnp.maximum(_dot(pooled, w1_ref[...]) + b1_ref[...], 0.0)
    z = jnp.maximum(_dot(z, w2_ref[...]) + b2_ref[...], 0.0)
    z = jnp.maximum(_dot(z, w3_ref[...]) + b3_ref[...], 0.0)
    o_ref[...] = _dot(z, w4_ref[...]) + b4_ref[...]


def kernel(x, edge_index, batch, W1, a_src1, a_dst1, b1, W2, a_src2, a_dst2, b2,
           gamma, beta, m1w, m1b, m2w, m2b, m3w, m3b, m4w, m4b):
    src, dst = edge_index[0], edge_index[1]
    src2 = jnp.pad(src.reshape(HR, 64), ((0, PADH - HR), (0, 0)))
    dst2 = jnp.pad(dst.reshape(HR, 64), ((0, PADH - HR), (0, 0)))

    acat1 = jnp.stack([a_src1, a_dst1], axis=1)
    acat2 = jnp.stack([a_src2, a_dst2], axis=1)

    h1, aa1 = pl.pallas_call(
        _tc_in_body,
        out_shape=(jax.ShapeDtypeStruct((N, D), jnp.float32),
                   jax.ShapeDtypeStruct((N, 2), jnp.float32)),
    )(x, W1, acat1)

    outp1, denomp1 = _sc_edge(h1, aa1[:, 0], aa1[:, 1], src2, dst2)

    h2, aa2 = pl.pallas_call(
        _tc_mid_body,
        out_shape=(jax.ShapeDtypeStruct((N, D), jnp.float32),
                   jax.ShapeDtypeStruct((N, 2), jnp.float32)),
    )(outp1, denomp1.T, b1.reshape(1, D), gamma.reshape(1, D),
      beta.reshape(1, D), W2, acat2)

    outp2, denomp2 = _sc_edge(h2, aa2[:, 0], aa2[:, 1], src2, dst2)

    z = pl.pallas_call(
        _tc_fin_body,
        out_shape=jax.ShapeDtypeStruct((G, m4w.shape[1]), jnp.float32),
    )(outp2, denomp2.T, b2.reshape(1, D), gamma.reshape(1, D),
      beta.reshape(1, D), batch.reshape(1, N),
      m1w, m1b.reshape(1, -1), m2w, m2b.reshape(1, -1),
      m3w, m3b.reshape(1, -1), m4w, m4b.reshape(1, -1))
    return z
